# baseline (speedup 1.0000x reference)
import jax
import jax.numpy as jnp
from jax import lax
from jax.experimental import pallas as pl
from jax.experimental.pallas import tpu as pltpu
from jax.experimental.pallas import tpu_sc as plsc

N = 10000
NP = 10240
E = 320000
D = 128
DH = 64
NC = 2
NS = 16
G = 128
GW = 160
NSUP = GW // 4
HGW = GW // 2
E_PAD = NS * GW * G
NROW = NS * GW

BN = 2048


def _mm_body(x_ref, w_ref, asv_ref, adv_ref, xw_ref, asr_ref, adr_ref):
    xw = jnp.dot(x_ref[...], w_ref[...], preferred_element_type=jnp.float32)
    xw_ref[0] = xw[:, :DH]
    xw_ref[1] = xw[:, DH:]
    asr_ref[...] = jnp.sum(xw * asv_ref[...][None, :], axis=1)
    adr_ref[...] = jnp.sum(xw * adv_ref[...][None, :], axis=1)


def _matmul(x, W, att_src, att_dst):
    grid = (N + BN - 1) // BN
    return pl.pallas_call(
        _mm_body,
        grid=(grid,),
        in_specs=[
            pl.BlockSpec((BN, D), lambda i: (i, 0)),
            pl.BlockSpec((D, D), lambda i: (0, 0)),
            pl.BlockSpec((D,), lambda i: (0,)),
            pl.BlockSpec((D,), lambda i: (0,)),
        ],
        out_specs=[
            pl.BlockSpec((NC, BN, DH), lambda i: (0, i, 0)),
            pl.BlockSpec((BN,), lambda i: (i,)),
            pl.BlockSpec((BN,), lambda i: (i,)),
        ],
        out_shape=[
            jax.ShapeDtypeStruct((NC, N, DH), jnp.float32),
            jax.ShapeDtypeStruct((N,), jnp.float32),
            jax.ShapeDtypeStruct((N,), jnp.float32),
        ],
    )(x, W, att_src, att_dst)


def _sc_body(asrc_h, adst_h, srcg_h, dstg_h, xw2_h, accp_h, s0_h, s1_h,
             asrc_v, adst_v, src_v, dst_v, sidx, didx,
             exg0, exg1, exg2, exg3, rows0, rows1, rows2, rows3, zs_v,
             acc_sh, s_sh,
             gsem0, gsem1, gsem2, gsem3, ssem0, ssem1, ssem2, ssem3,
             tsem0, tsem1, tsem2, tsem3):
    cid = lax.axis_index("c")
    sid = lax.axis_index("s")
    exg = [exg0, exg1, exg2, exg3]
    rows = [rows0, rows1, rows2, rows3]
    gsem = [gsem0, gsem1, gsem2, gsem3]
    ssem = [ssem0, ssem1, ssem2, ssem3]
    tsem = [tsem0, tsem1, tsem2, tsem3]

    def stage_sidx(r, b):
        for k in range(8):
            sidx[b, pl.ds(k * 16, 16)] = src_v[r, pl.ds(k * 16, 16)]

    def stage_didx(r, b):
        for k in range(8):
            didx[b, pl.ds(k * 16, 16)] = dst_v[r, pl.ds(k * 16, 16)]

    def issue_gather(b):
        pltpu.async_copy(xw2_h.at[cid].at[sidx.at[b]], rows[b], gsem[b])

    def wait_gather(b):
        pltpu.make_async_copy(xw2_h.at[cid].at[sidx.at[0]], rows[b],
                              gsem[b]).wait()

    def wait_scatter(b):
        pltpu.make_async_copy(rows[b], acc_sh.at[didx.at[0]], ssem[b]).wait()

    def wait_ssc(b):
        pltpu.make_async_copy(exg[b], s_sh.at[didx.at[0]], tsem[b]).wait()

    pltpu.sync_copy(srcg_h.at[pl.ds(sid * GW, HGW)], src_v)
    pltpu.sync_copy(dstg_h.at[pl.ds(sid * GW, HGW)], dst_v)
    stage_sidx(0, 0)
    stage_sidx(1, 1)
    issue_gather(0)
    issue_gather(1)

    pltpu.sync_copy(asrc_h, asrc_v)
    pltpu.sync_copy(adst_h, adst_v)

    z16 = jnp.zeros((16,), jnp.float32)

    def _zrow(j, c):
        for k in range(DH // 16):
            rows2[j, pl.ds(k * 16, 16)] = z16
        return c
    lax.fori_loop(0, G, _zrow, 0)

    def _zs(j, c):
        zs_v[pl.ds(j * 16, 16)] = z16
        return c
    lax.fori_loop(0, 40, _zs, 0)

    rbase = sid * 640
    for i in range(5):
        pltpu.sync_copy(rows2, acc_sh.at[pl.ds(rbase + i * G, G)])
    pltpu.sync_copy(zs_v, s_sh.at[pl.ds(sid * 640, 640)])
    plsc.subcore_barrier()

    def _super(it, c):
        @pl.when(it == NSUP // 2)
        def _():
            wait_scatter(0)
            wait_scatter(1)
            pltpu.sync_copy(srcg_h.at[pl.ds(sid * GW + HGW, HGW)], src_v)
            pltpu.sync_copy(dstg_h.at[pl.ds(sid * GW + HGW, HGW)], dst_v)
            stage_sidx(0, 0)
            stage_sidx(1, 1)
            issue_gather(0)
            issue_gather(1)

        for b in range(4):
            q = it * 4 + b
            bb = (b + 2) % 4
            r = jnp.where(q >= HGW, q - HGW, q)
            if b < 2:
                @pl.when(it > 0)
                def _():
                    wait_scatter(bb)
                stage_sidx(r + 2, bb)
                issue_gather(bb)
            else:
                @pl.when(jnp.logical_and(it < NSUP - 1,
                                         it != NSUP // 2 - 1))
                def _():
                    wait_scatter(bb)
                    stage_sidx(r + 2, bb)
                    issue_gather(bb)

            @pl.when(jnp.logical_and(cid == b % 2, it > 0))
            def _():
                wait_ssc(b)
            for k in range(8):
                si = src_v[r, pl.ds(k * 16, 16)]
                di = dst_v[r, pl.ds(k * 16, 16)]
                e = (plsc.load_gather(asrc_v, [si])
                     + plsc.load_gather(adst_v, [di]))
                e = jnp.where(e > 0, e, e * jnp.float32(0.2))
                ex = jnp.exp(e)
                gid = (sid * GW + q) * G + k * 16 + lax.iota(jnp.int32, 16)
                ex = jnp.where(gid < E, ex, jnp.float32(0.0))
                exg[b][pl.ds(k * 16, 16)] = ex

            wait_gather(b)
            rb = rows[b]
            eb = exg[b]

            def _scl(jj, c2):
                sub = rb.at[pl.ds(jj * 16, 16)]
                ex16 = eb[pl.ds(jj * 16, 16)]
                for u in range(16):
                    exj = lax.gather(
                        ex16, jnp.full((16, 1), u, jnp.int32),
                        lax.GatherDimensionNumbers(
                            offset_dims=(), collapsed_slice_dims=(0,),
                            start_index_map=(0,)),
                        (1,), mode=lax.GatherScatterMode.PROMISE_IN_BOUNDS)
                    for k in range(DH // 16):
                        sub[u, pl.ds(k * 16, 16)] = (
                            sub[u, pl.ds(k * 16, 16)] * exj)
                return c2
            lax.fori_loop(0, G // 16, _scl, 0)

            stage_didx(r, b)
            pltpu.async_copy(rb, acc_sh.at[didx.at[b]], ssem[b], add=True)

            @pl.when(cid == b % 2)
            def _():
                pltpu.async_copy(eb, s_sh.at[didx.at[b]], tsem[b], add=True)
        return c
    lax.fori_loop(0, NSUP, _super, 0)

    for b in range(4):
        wait_scatter(b)

        @pl.when(cid == b % 2)
        def _():
            wait_ssc(b)

    plsc.subcore_barrier()
    pltpu.sync_copy(acc_sh.at[pl.ds(rbase, 640)],
                    accp_h.at[cid, pl.ds(rbase, 640)])

    @pl.when(jnp.logical_and(sid == 0, cid == 0))
    def _():
        pltpu.sync_copy(s_sh, s0_h)

    @pl.when(jnp.logical_and(sid == 0, cid == 1))
    def _():
        pltpu.sync_copy(s_sh, s1_h)


def _sc_edge(asrc, adst, src_g, dst_g, xw2):
    mesh = plsc.VectorSubcoreMesh(core_axis_name="c", subcore_axis_name="s",
                                  num_cores=NC, num_subcores=NS)
    f = pl.kernel(
        _sc_body,
        out_type=[
            jax.ShapeDtypeStruct((NC, NP, DH), jnp.float32),
            jax.ShapeDtypeStruct((NP,), jnp.float32),
            jax.ShapeDtypeStruct((NP,), jnp.float32),
        ],
        mesh=mesh,
        compiler_params=pltpu.CompilerParams(use_tc_tiling_on_sc=False,
                                             needs_layout_passes=False),
        scratch_types=(
            [pltpu.VMEM((N,), jnp.float32)] * 2
            + [pltpu.VMEM((HGW, G), jnp.int32)] * 2
            + [pltpu.VMEM((4, G), jnp.int32)] * 2
            + [pltpu.VMEM((G,), jnp.float32)] * 4
            + [pltpu.VMEM((G, DH), jnp.float32)] * 4
            + [pltpu.VMEM((640,), jnp.float32)]
            + [pltpu.VMEM_SHARED((NP, DH), jnp.float32),
               pltpu.VMEM_SHARED((NP,), jnp.float32)]
            + [pltpu.SemaphoreType.DMA] * 12
        ),
    )
    return f(asrc, adst, src_g, dst_g, xw2)


def _ep_body(accp_ref, s0_ref, s1_ref, b_ref, o_ref):
    a = jnp.concatenate([accp_ref[0], accp_ref[1]], axis=1)
    s = s0_ref[...] + s1_ref[...]
    safe = jnp.where(s == 0, jnp.float32(1.0), s)
    o_ref[...] = a / safe[:, None] + b_ref[...][None, :]


def _epilogue(accp, s0, s1, bias):
    grid = (N + BN - 1) // BN
    return pl.pallas_call(
        _ep_body,
        grid=(grid,),
        in_specs=[
            pl.BlockSpec((NC, BN, DH), lambda i: (0, i, 0)),
            pl.BlockSpec((BN,), lambda i: (i,)),
            pl.BlockSpec((BN,), lambda i: (i,)),
            pl.BlockSpec((D,), lambda i: (0,)),
        ],
        out_specs=pl.BlockSpec((BN, D), lambda i: (i, 0)),
        out_shape=jax.ShapeDtypeStruct((N, D), jnp.float32),
    )(accp, s0, s1, bias)


def kernel(x, edge_index, edge_attr, h, batch, W, att_src, att_dst, bias):
    src = edge_index[0].astype(jnp.int32)
    dst = edge_index[1].astype(jnp.int32)
    src_g = jnp.pad(src, (0, E_PAD - E)).reshape(NROW, G)
    dst_g = jnp.pad(dst, (0, E_PAD - E)).reshape(NROW, G)

    xw2, asrc, adst = _matmul(x, W, att_src, att_dst)
    accp, s0, s1 = _sc_edge(asrc, adst, src_g, dst_g, xw2)
    return _epilogue(accp, s0, s1, bias)


# speedup vs baseline: 1.0546x; 1.0546x over previous
import jax
import jax.numpy as jnp
from jax import lax
from jax.experimental import pallas as pl
from jax.experimental.pallas import tpu as pltpu
from jax.experimental.pallas import tpu_sc as plsc

N = 10000
NP = 10240
E = 320000
D = 128
DH = 64
NC = 2
NS = 16
G = 128
GW = 160
NSUP = GW // 4
HGW = GW // 2
E_PAD = NS * GW * G
NROW = NS * GW

BN = 2048


def _mm_body(x_ref, w_ref, asv_ref, adv_ref, xw_ref, asr_ref, adr_ref):
    xw = jnp.dot(x_ref[...], w_ref[...], preferred_element_type=jnp.float32)
    xw_ref[0] = xw[:, :DH]
    xw_ref[1] = xw[:, DH:]
    asr_ref[...] = jnp.sum(xw * asv_ref[...][None, :], axis=1)
    adr_ref[...] = jnp.sum(xw * adv_ref[...][None, :], axis=1)


def _matmul(x, W, att_src, att_dst):
    grid = (N + BN - 1) // BN
    return pl.pallas_call(
        _mm_body,
        grid=(grid,),
        in_specs=[
            pl.BlockSpec((BN, D), lambda i: (i, 0)),
            pl.BlockSpec((D, D), lambda i: (0, 0)),
            pl.BlockSpec((D,), lambda i: (0,)),
            pl.BlockSpec((D,), lambda i: (0,)),
        ],
        out_specs=[
            pl.BlockSpec((NC, BN, DH), lambda i: (0, i, 0)),
            pl.BlockSpec((BN,), lambda i: (i,)),
            pl.BlockSpec((BN,), lambda i: (i,)),
        ],
        out_shape=[
            jax.ShapeDtypeStruct((NC, N, DH), jnp.float32),
            jax.ShapeDtypeStruct((N,), jnp.float32),
            jax.ShapeDtypeStruct((N,), jnp.float32),
        ],
    )(x, W, att_src, att_dst)


def _sc_body(asrc_h, adst_h, srcg_h, dstg_h, xw2_h, accp_h, s0_h, s1_h,
             asrc_v, adst_v, src_v, dst_v, sidx, didx,
             exg0, exg1, exg2, exg3, rows0, rows1, rows2, rows3, zs_v,
             acc_sh, s_sh,
             gsem0, gsem1, gsem2, gsem3, ssem0, ssem1, ssem2, ssem3,
             tsem0, tsem1, tsem2, tsem3):
    cid = lax.axis_index("c")
    sid = lax.axis_index("s")
    exg = [exg0, exg1, exg2, exg3]
    rows = [rows0, rows1, rows2, rows3]
    gsem = [gsem0, gsem1, gsem2, gsem3]
    ssem = [ssem0, ssem1, ssem2, ssem3]
    tsem = [tsem0, tsem1, tsem2, tsem3]

    def stage_sidx(r, b):
        for k in range(8):
            sidx[b, pl.ds(k * 16, 16)] = src_v[r, pl.ds(k * 16, 16)]

    def stage_didx(r, b):
        for k in range(8):
            didx[b, pl.ds(k * 16, 16)] = dst_v[r, pl.ds(k * 16, 16)]

    def issue_gather(b):
        pltpu.async_copy(xw2_h.at[cid].at[sidx.at[b]], rows[b], gsem[b])

    def wait_gather(b):
        pltpu.make_async_copy(xw2_h.at[cid].at[sidx.at[0]], rows[b],
                              gsem[b]).wait()

    def wait_scatter(b):
        pltpu.make_async_copy(rows[b], acc_sh.at[didx.at[0]], ssem[b]).wait()

    def wait_ssc(b):
        pltpu.make_async_copy(exg[b], s_sh.at[didx.at[0]], tsem[b]).wait()

    pltpu.sync_copy(srcg_h.at[pl.ds(sid * GW, HGW)], src_v)
    pltpu.sync_copy(dstg_h.at[pl.ds(sid * GW, HGW)], dst_v)
    stage_sidx(0, 0)
    stage_sidx(1, 1)
    issue_gather(0)
    issue_gather(1)

    pltpu.sync_copy(asrc_h, asrc_v)
    pltpu.sync_copy(adst_h, adst_v)

    z16 = jnp.zeros((16,), jnp.float32)

    def _zrow(j, c):
        for k in range(DH // 16):
            rows2[j, pl.ds(k * 16, 16)] = z16
        return c
    lax.fori_loop(0, G, _zrow, 0)

    def _zs(j, c):
        zs_v[pl.ds(j * 16, 16)] = z16
        return c
    lax.fori_loop(0, 40, _zs, 0)

    rbase = sid * 640
    for i in range(5):
        pltpu.sync_copy(rows2, acc_sh.at[pl.ds(rbase + i * G, G)])
    pltpu.sync_copy(zs_v, s_sh.at[pl.ds(sid * 640, 640)])
    plsc.subcore_barrier()

    def _super(it, c):
        @pl.when(it == NSUP // 2)
        def _():
            wait_scatter(0)
            wait_scatter(1)
            pltpu.sync_copy(srcg_h.at[pl.ds(sid * GW + HGW, HGW)], src_v)
            pltpu.sync_copy(dstg_h.at[pl.ds(sid * GW + HGW, HGW)], dst_v)
            stage_sidx(0, 0)
            stage_sidx(1, 1)
            issue_gather(0)
            issue_gather(1)

        for b in range(4):
            q = it * 4 + b
            bb = (b + 2) % 4
            r = jnp.where(q >= HGW, q - HGW, q)
            if b < 2:
                @pl.when(it > 0)
                def _():
                    wait_scatter(bb)
                stage_sidx(r + 2, bb)
                issue_gather(bb)
            else:
                @pl.when(jnp.logical_and(it < NSUP - 1,
                                         it != NSUP // 2 - 1))
                def _():
                    wait_scatter(bb)
                    stage_sidx(r + 2, bb)
                    issue_gather(bb)

            @pl.when(jnp.logical_and(cid == b % 2, it > 0))
            def _():
                wait_ssc(b)
            for k in range(8):
                si = src_v[r, pl.ds(k * 16, 16)]
                di = dst_v[r, pl.ds(k * 16, 16)]
                e = (plsc.load_gather(asrc_v, [si])
                     + plsc.load_gather(adst_v, [di]))
                e = jnp.where(e > 0, e, e * jnp.float32(0.2))
                ex = jnp.exp(e)
                gid = (sid * GW + q) * G + k * 16 + lax.iota(jnp.int32, 16)
                ex = jnp.where(gid < E, ex, jnp.float32(0.0))
                exg[b][pl.ds(k * 16, 16)] = ex

            wait_gather(b)
            rb = rows[b]
            eb = exg[b]

            def _scl(jj, c2):
                sub = rb.at[pl.ds(jj * 16, 16)]
                for u in range(16):
                    exj = plsc.load_gather(
                        eb, [jnp.full((16,), jj * 16 + u, jnp.int32)])
                    for k in range(DH // 16):
                        sub[u, pl.ds(k * 16, 16)] = (
                            sub[u, pl.ds(k * 16, 16)] * exj)
                return c2
            lax.fori_loop(0, G // 16, _scl, 0)

            stage_didx(r, b)
            pltpu.async_copy(rb, acc_sh.at[didx.at[b]], ssem[b], add=True)

            @pl.when(cid == b % 2)
            def _():
                pltpu.async_copy(eb, s_sh.at[didx.at[b]], tsem[b], add=True)
        return c
    lax.fori_loop(0, NSUP, _super, 0)

    for b in range(4):
        wait_scatter(b)

        @pl.when(cid == b % 2)
        def _():
            wait_ssc(b)

    plsc.subcore_barrier()
    pltpu.sync_copy(acc_sh.at[pl.ds(rbase, 640)],
                    accp_h.at[cid, pl.ds(rbase, 640)])

    @pl.when(jnp.logical_and(sid == 0, cid == 0))
    def _():
        pltpu.sync_copy(s_sh, s0_h)

    @pl.when(jnp.logical_and(sid == 0, cid == 1))
    def _():
        pltpu.sync_copy(s_sh, s1_h)


def _sc_edge(asrc, adst, src_g, dst_g, xw2):
    mesh = plsc.VectorSubcoreMesh(core_axis_name="c", subcore_axis_name="s",
                                  num_cores=NC, num_subcores=NS)
    f = pl.kernel(
        _sc_body,
        out_type=[
            jax.ShapeDtypeStruct((NC, NP, DH), jnp.float32),
            jax.ShapeDtypeStruct((NP,), jnp.float32),
            jax.ShapeDtypeStruct((NP,), jnp.float32),
        ],
        mesh=mesh,
        compiler_params=pltpu.CompilerParams(use_tc_tiling_on_sc=False,
                                             needs_layout_passes=False),
        scratch_types=(
            [pltpu.VMEM((N,), jnp.float32)] * 2
            + [pltpu.VMEM((HGW, G), jnp.int32)] * 2
            + [pltpu.VMEM((4, G), jnp.int32)] * 2
            + [pltpu.VMEM((G,), jnp.float32)] * 4
            + [pltpu.VMEM((G, DH), jnp.float32)] * 4
            + [pltpu.VMEM((640,), jnp.float32)]
            + [pltpu.VMEM_SHARED((NP, DH), jnp.float32),
               pltpu.VMEM_SHARED((NP,), jnp.float32)]
            + [pltpu.SemaphoreType.DMA] * 12
        ),
    )
    return f(asrc, adst, src_g, dst_g, xw2)


def _ep_body(accp_ref, s0_ref, s1_ref, b_ref, o_ref):
    a = jnp.concatenate([accp_ref[0], accp_ref[1]], axis=1)
    s = s0_ref[...] + s1_ref[...]
    safe = jnp.where(s == 0, jnp.float32(1.0), s)
    o_ref[...] = a / safe[:, None] + b_ref[...][None, :]


def _epilogue(accp, s0, s1, bias):
    grid = (N + BN - 1) // BN
    return pl.pallas_call(
        _ep_body,
        grid=(grid,),
        in_specs=[
            pl.BlockSpec((NC, BN, DH), lambda i: (0, i, 0)),
            pl.BlockSpec((BN,), lambda i: (i,)),
            pl.BlockSpec((BN,), lambda i: (i,)),
            pl.BlockSpec((D,), lambda i: (0,)),
        ],
        out_specs=pl.BlockSpec((BN, D), lambda i: (i, 0)),
        out_shape=jax.ShapeDtypeStruct((N, D), jnp.float32),
    )(accp, s0, s1, bias)


def kernel(x, edge_index, edge_attr, h, batch, W, att_src, att_dst, bias):
    src = edge_index[0].astype(jnp.int32)
    dst = edge_index[1].astype(jnp.int32)
    src_g = jnp.pad(src, (0, E_PAD - E)).reshape(NROW, G)
    dst_g = jnp.pad(dst, (0, E_PAD - E)).reshape(NROW, G)

    xw2, asrc, adst = _matmul(x, W, att_src, att_dst)
    accp, s0, s1 = _sc_edge(asrc, adst, src_g, dst_g, xw2)
    return _epilogue(accp, s0, s1, bias)


# speedup vs baseline: 1.4297x; 1.3558x over previous
import jax
import jax.numpy as jnp
from jax import lax
from jax.experimental import pallas as pl
from jax.experimental.pallas import tpu as pltpu
from jax.experimental.pallas import tpu_sc as plsc

N = 10000
NP = 10240
E = 320000
D = 128
DH = 64
NC = 2
NS = 16
G = 128
GW = 160
NSUP = GW // 4
HGW = GW // 2
E_PAD = NS * GW * G
NROW = NS * GW

BN = 2048


def _mm_body(x_ref, w_ref, asv_ref, adv_ref, xw_ref, asr_ref, adr_ref):
    xw = jnp.dot(x_ref[...], w_ref[...], preferred_element_type=jnp.float32)
    xw_ref[0] = xw[:, :DH]
    xw_ref[1] = xw[:, DH:]
    asr_ref[...] = jnp.sum(xw * asv_ref[...][None, :], axis=1)
    adr_ref[...] = jnp.sum(xw * adv_ref[...][None, :], axis=1)


def _matmul(x, W, att_src, att_dst):
    grid = (N + BN - 1) // BN
    return pl.pallas_call(
        _mm_body,
        grid=(grid,),
        in_specs=[
            pl.BlockSpec((BN, D), lambda i: (i, 0)),
            pl.BlockSpec((D, D), lambda i: (0, 0)),
            pl.BlockSpec((D,), lambda i: (0,)),
            pl.BlockSpec((D,), lambda i: (0,)),
        ],
        out_specs=[
            pl.BlockSpec((NC, BN, DH), lambda i: (0, i, 0)),
            pl.BlockSpec((BN,), lambda i: (i,)),
            pl.BlockSpec((BN,), lambda i: (i,)),
        ],
        out_shape=[
            jax.ShapeDtypeStruct((NC, N, DH), jnp.float32),
            jax.ShapeDtypeStruct((N,), jnp.float32),
            jax.ShapeDtypeStruct((N,), jnp.float32),
        ],
    )(x, W, att_src, att_dst)


def _sc_body(asrc_h, adst_h, srcg_h, dstg_h, xw2_h, accp_h, s0_h, s1_h,
             asrc_v, adst_v, src_v, dst_v, sidx, didx,
             exg0, exg1, exg2, exg3, rows0, rows1, rows2, rows3, zs_v,
             acc_sh, s_sh,
             gsem0, gsem1, gsem2, gsem3, ssem0, ssem1, ssem2, ssem3,
             tsem0, tsem1, tsem2, tsem3):
    cid = lax.axis_index("c")
    sid = lax.axis_index("s")
    exg = [exg0, exg1, exg2, exg3]
    rows = [rows0, rows1, rows2, rows3]
    gsem = [gsem0, gsem1, gsem2, gsem3]
    ssem = [ssem0, ssem1, ssem2, ssem3]
    tsem = [tsem0, tsem1, tsem2, tsem3]

    def stage_sidx(r, b):
        for k in range(8):
            sidx[b, pl.ds(k * 16, 16)] = src_v[r, pl.ds(k * 16, 16)]

    def stage_didx(r, b):
        for k in range(8):
            didx[b, pl.ds(k * 16, 16)] = dst_v[r, pl.ds(k * 16, 16)]

    def issue_gather(b):
        pltpu.async_copy(xw2_h.at[cid].at[sidx.at[b]], rows[b], gsem[b])

    def wait_gather(b):
        pltpu.make_async_copy(xw2_h.at[cid].at[sidx.at[0]], rows[b],
                              gsem[b]).wait()

    def wait_scatter(b):
        pltpu.make_async_copy(rows[b], acc_sh.at[didx.at[0]], ssem[b]).wait()

    def wait_ssc(b):
        pltpu.make_async_copy(exg[b], s_sh.at[didx.at[0]], tsem[b]).wait()

    pltpu.sync_copy(srcg_h.at[pl.ds(sid * GW, HGW)], src_v)
    pltpu.sync_copy(dstg_h.at[pl.ds(sid * GW, HGW)], dst_v)
    stage_sidx(0, 0)
    stage_sidx(1, 1)
    issue_gather(0)
    issue_gather(1)

    pltpu.sync_copy(asrc_h, asrc_v)
    pltpu.sync_copy(adst_h, adst_v)

    z16 = jnp.zeros((16,), jnp.float32)

    def _zrow(j, c):
        for k in range(DH // 16):
            rows2[j, pl.ds(k * 16, 16)] = z16
        return c
    lax.fori_loop(0, G, _zrow, 0)

    def _zs(j, c):
        zs_v[pl.ds(j * 16, 16)] = z16
        return c
    lax.fori_loop(0, 40, _zs, 0)

    rbase = sid * 640
    for i in range(5):
        pltpu.sync_copy(rows2, acc_sh.at[pl.ds(rbase + i * G, G)])
    pltpu.sync_copy(zs_v, s_sh.at[pl.ds(sid * 640, 640)])
    plsc.subcore_barrier()

    def _super(it, c):
        @pl.when(it == NSUP // 2)
        def _():
            wait_scatter(0)
            wait_scatter(1)
            pltpu.sync_copy(srcg_h.at[pl.ds(sid * GW + HGW, HGW)], src_v)
            pltpu.sync_copy(dstg_h.at[pl.ds(sid * GW + HGW, HGW)], dst_v)
            stage_sidx(0, 0)
            stage_sidx(1, 1)
            issue_gather(0)
            issue_gather(1)

        for b in range(4):
            q = it * 4 + b
            bb = (b + 2) % 4
            r = jnp.where(q >= HGW, q - HGW, q)
            if b < 2:
                @pl.when(it > 0)
                def _():
                    wait_scatter(bb)
                stage_sidx(r + 2, bb)
                issue_gather(bb)
            else:
                @pl.when(jnp.logical_and(it < NSUP - 1,
                                         it != NSUP // 2 - 1))
                def _():
                    wait_scatter(bb)
                    stage_sidx(r + 2, bb)
                    issue_gather(bb)

            @pl.when(jnp.logical_and(cid == b % 2, it > 0))
            def _():
                wait_ssc(b)
            for k in range(8):
                si = src_v[r, pl.ds(k * 16, 16)]
                di = dst_v[r, pl.ds(k * 16, 16)]
                e = (plsc.load_gather(asrc_v, [si])
                     + plsc.load_gather(adst_v, [di]))
                e = jnp.where(e > 0, e, e * jnp.float32(0.2))
                ex = jnp.exp(e)
                gid = (sid * GW + q) * G + k * 16 + lax.iota(jnp.int32, 16)
                ex = jnp.where(gid < E, ex, jnp.float32(0.0))
                exg[b][pl.ds(k * 16, 16)] = ex

            wait_gather(b)
            rb = rows[b]
            eb = exg[b]

            @plsc.parallel_loop(0, G, unroll=4)
            def _scl(j):
                exj = plsc.load_gather(
                    eb, [jnp.full((16,), j, jnp.int32)])
                for k in range(DH // 16):
                    rb[j, pl.ds(k * 16, 16)] = (
                        rb[j, pl.ds(k * 16, 16)] * exj)

            stage_didx(r, b)
            pltpu.async_copy(rb, acc_sh.at[didx.at[b]], ssem[b], add=True)

            @pl.when(cid == b % 2)
            def _():
                pltpu.async_copy(eb, s_sh.at[didx.at[b]], tsem[b], add=True)
        return c
    lax.fori_loop(0, NSUP, _super, 0)

    for b in range(4):
        wait_scatter(b)

        @pl.when(cid == b % 2)
        def _():
            wait_ssc(b)

    plsc.subcore_barrier()
    pltpu.sync_copy(acc_sh.at[pl.ds(rbase, 640)],
                    accp_h.at[cid, pl.ds(rbase, 640)])

    @pl.when(jnp.logical_and(sid == 0, cid == 0))
    def _():
        pltpu.sync_copy(s_sh, s0_h)

    @pl.when(jnp.logical_and(sid == 0, cid == 1))
    def _():
        pltpu.sync_copy(s_sh, s1_h)


def _sc_edge(asrc, adst, src_g, dst_g, xw2):
    mesh = plsc.VectorSubcoreMesh(core_axis_name="c", subcore_axis_name="s",
                                  num_cores=NC, num_subcores=NS)
    f = pl.kernel(
        _sc_body,
        out_type=[
            jax.ShapeDtypeStruct((NC, NP, DH), jnp.float32),
            jax.ShapeDtypeStruct((NP,), jnp.float32),
            jax.ShapeDtypeStruct((NP,), jnp.float32),
        ],
        mesh=mesh,
        compiler_params=pltpu.CompilerParams(use_tc_tiling_on_sc=False,
                                             needs_layout_passes=False),
        scratch_types=(
            [pltpu.VMEM((N,), jnp.float32)] * 2
            + [pltpu.VMEM((HGW, G), jnp.int32)] * 2
            + [pltpu.VMEM((4, G), jnp.int32)] * 2
            + [pltpu.VMEM((G,), jnp.float32)] * 4
            + [pltpu.VMEM((G, DH), jnp.float32)] * 4
            + [pltpu.VMEM((640,), jnp.float32)]
            + [pltpu.VMEM_SHARED((NP, DH), jnp.float32),
               pltpu.VMEM_SHARED((NP,), jnp.float32)]
            + [pltpu.SemaphoreType.DMA] * 12
        ),
    )
    return f(asrc, adst, src_g, dst_g, xw2)


def _ep_body(accp_ref, s0_ref, s1_ref, b_ref, o_ref):
    a = jnp.concatenate([accp_ref[0], accp_ref[1]], axis=1)
    s = s0_ref[...] + s1_ref[...]
    safe = jnp.where(s == 0, jnp.float32(1.0), s)
    o_ref[...] = a / safe[:, None] + b_ref[...][None, :]


def _epilogue(accp, s0, s1, bias):
    grid = (N + BN - 1) // BN
    return pl.pallas_call(
        _ep_body,
        grid=(grid,),
        in_specs=[
            pl.BlockSpec((NC, BN, DH), lambda i: (0, i, 0)),
            pl.BlockSpec((BN,), lambda i: (i,)),
            pl.BlockSpec((BN,), lambda i: (i,)),
            pl.BlockSpec((D,), lambda i: (0,)),
        ],
        out_specs=pl.BlockSpec((BN, D), lambda i: (i, 0)),
        out_shape=jax.ShapeDtypeStruct((N, D), jnp.float32),
    )(accp, s0, s1, bias)


def kernel(x, edge_index, edge_attr, h, batch, W, att_src, att_dst, bias):
    src = edge_index[0].astype(jnp.int32)
    dst = edge_index[1].astype(jnp.int32)
    src_g = jnp.pad(src, (0, E_PAD - E)).reshape(NROW, G)
    dst_g = jnp.pad(dst, (0, E_PAD - E)).reshape(NROW, G)

    xw2, asrc, adst = _matmul(x, W, att_src, att_dst)
    accp, s0, s1 = _sc_edge(asrc, adst, src_g, dst_g, xw2)
    return _epilogue(accp, s0, s1, bias)
